# Initial kernel scaffold; baseline (speedup 1.0000x reference)
import jax
import jax.numpy as jnp
from jax import lax
from jax.experimental import pallas as pl
from jax.experimental.pallas import tpu as pltpu
from jax.experimental.pallas import tpu_sc as plsc

N_POINTS = 160000
N_FRAMES = 10000
N_SEQS = 128
FEAT = 256
NUM_SB = 8
LANE = 16
NFG = FEAT // LANE
NC = 2
NS = 16
NW = NC * NS


def _seg_max_sc(n_items: int, n_out_pad: int, opw: int, ch: int):
  assert n_out_pad == NW * opw

  def body(x_hbm, ids_hbm, bounds_hbm, out_hbm, xbuf, idbuf, bbuf, outbuf):
    w = lax.axis_index("s") * NC + lax.axis_index("c")
    fbase = w * opw
    pltpu.sync_copy(bounds_hbm, bbuf)
    lane = lax.iota(jnp.int32, 16)
    minus1 = jnp.full((16,), -1, jnp.int32)
    ninf = jnp.full((16,), -jnp.inf, jnp.float32)

    def extract(idx):
      r = jnp.int32(-1)
      for k in range(3):
        v = bbuf[pl.ds(k * 16, 16)]
        r = jnp.maximum(r, jnp.max(jnp.where(lane + (k * 16) == idx, v, minus1)))
      return r

    start = extract(w)
    end = extract(w + 1)
    astart = (start // 16) * 16
    nch = jnp.maximum((end - astart + ch - 1) // ch, 0)

    def init_body(i, _):
      outbuf[pl.ds(pl.multiple_of(i * 16, 16), 16)] = ninf
      return 0
    lax.fori_loop(0, opw * NFG, init_body, 0)

    def flush(cur_id, acc):
      roff = pl.multiple_of((cur_id - fbase) * FEAT, FEAT)
      for f in range(NFG):
        o = pl.multiple_of(roff + f * 16, 16)
        outbuf[pl.ds(o, 16)] = jnp.maximum(outbuf[pl.ds(o, 16)], acc[f])

    def chunk_body(c, carry):
      lo = astart + c * ch
      cbase = pl.multiple_of((jnp.minimum(lo, n_items - ch) // 16) * 16, 16)
      pltpu.sync_copy(x_hbm.at[pl.ds(pl.multiple_of(cbase * FEAT, 16), ch * FEAT)],
                      xbuf)
      pltpu.sync_copy(ids_hbm.at[pl.ds(cbase, ch)], idbuf)

      def point_body(j, pcarry):
        cur_id, acc = pcarry
        p = cbase + j
        valid = jnp.logical_and(p >= start, p < end)
        goff = pl.multiple_of((j // 16) * 16, 16)
        idsv = idbuf[pl.ds(goff, 16)]
        idp = jnp.max(jnp.where(lane == (j % 16), idsv, minus1))
        do_flush = jnp.logical_and(valid,
                                   jnp.logical_and(cur_id >= 0, idp != cur_id))

        @pl.when(do_flush)
        def _():
          flush(cur_id, acc)

        same = jnp.logical_and(valid, idp == cur_id)
        first = jnp.logical_and(valid, idp != cur_id)
        xoff = pl.multiple_of(j * FEAT, FEAT)
        nacc = []
        for f in range(NFG):
          row = xbuf[pl.ds(xoff + f * 16, 16)]
          a = jnp.where(same, jnp.maximum(acc[f], row),
                        jnp.where(first, row, acc[f]))
          nacc.append(a)
        ncur = jnp.where(valid, idp, cur_id)
        return ncur, tuple(nacc)

      return lax.fori_loop(0, ch, point_body, carry)

    cur_id, acc = lax.fori_loop(0, nch, chunk_body,
                                (jnp.int32(-1), (ninf,) * NFG))

    @pl.when(cur_id >= 0)
    def _():
      flush(cur_id, acc)

    pltpu.sync_copy(outbuf, out_hbm.at[pl.ds(fbase * FEAT, opw * FEAT)])

  mesh = plsc.VectorSubcoreMesh(core_axis_name="c", subcore_axis_name="s")
  return pl.kernel(
      body,
      out_type=jax.ShapeDtypeStruct((n_out_pad * FEAT,), jnp.float32),
      mesh=mesh,
      scratch_types=[
          pltpu.VMEM((ch * FEAT,), jnp.float32),
          pltpu.VMEM((ch,), jnp.int32),
          pltpu.VMEM((48,), jnp.int32),
          pltpu.VMEM((opw * FEAT,), jnp.float32),
      ],
  )


def _heads_tc(fp_ref, sp_ref, bbc_ref, cosr_ref, sinr_ref, swap_ref,
              Wc_ref, bc_ref, Wv_ref, bv_ref, Wy_ref, by_ref,
              Wsb_ref, bsb_ref, Wsr_ref, bsr_ref,
              cen_ref, vel_ref, yaw_ref, sr_ref, sb_ref):
  fp = fp_ref[...]
  hp = jax.lax.Precision.HIGHEST
  cen_ref[...] = (jnp.dot(fp, Wc_ref[...], precision=hp,
                          preferred_element_type=jnp.float32)
                  + bc_ref[...] + bbc_ref[...])
  vel_ref[...] = (jnp.dot(fp, Wv_ref[...], precision=hp,
                          preferred_element_type=jnp.float32) + bv_ref[...])
  y = (jnp.dot(fp, Wy_ref[...], precision=hp,
               preferred_element_type=jnp.float32) + by_ref[...])
  ya = jnp.dot(y, swap_ref[...], precision=hp,
               preferred_element_type=jnp.float32)
  yaw_ref[...] = cosr_ref[...] * y + sinr_ref[...] * ya
  sp = sp_ref[...]
  sr_ref[...] = (jnp.dot(sp, Wsr_ref[...], precision=hp,
                         preferred_element_type=jnp.float32) + bsr_ref[...])
  logits = (jnp.dot(sp, Wsb_ref[...], precision=hp,
                    preferred_element_type=jnp.float32) + bsb_ref[...])
  m = jnp.max(logits, axis=1, keepdims=True)
  e = jnp.exp(logits - m)
  sb_ref[...] = e / jnp.sum(e, axis=1, keepdims=True)


def kernel(x, bbox_center, bbox_cos_yaw, bbox_sin_yaw, raw_xyz,
           point2frameidx, frame2batchidx,
           Wc, bc, Wy, by, Wv, bv, Wsb, bsb, Wsr, bsr):
  del raw_xyz
  opw1 = 320
  nfp = NW * opw1
  opw2 = N_SEQS // NW

  qs1 = (jnp.arange(NW + 1, dtype=jnp.int32) * opw1).clip(0, N_FRAMES)
  b1 = jnp.searchsorted(point2frameidx, qs1).astype(jnp.int32)
  b1 = jnp.concatenate([b1, jnp.full((48 - (NW + 1),), N_POINTS, jnp.int32)])

  qs2 = jnp.arange(NW + 1, dtype=jnp.int32) * opw2
  b2 = jnp.searchsorted(frame2batchidx, qs2).astype(jnp.int32)
  b2 = jnp.concatenate([b2, jnp.full((48 - (NW + 1),), N_FRAMES, jnp.int32)])

  seg1 = _seg_max_sc(N_POINTS, nfp, opw1, 64)
  fp_flat = seg1(x.reshape(-1), point2frameidx, b1)

  seg2 = _seg_max_sc(N_FRAMES, N_SEQS, opw2, 64)
  sp_flat = seg2(fp_flat, frame2batchidx, b2)

  fp = fp_flat.reshape(nfp, FEAT)[:N_FRAMES]
  sp = sp_flat.reshape(N_SEQS, FEAT)

  swap = jnp.array([[0.0, -1.0], [1.0, 0.0]], dtype=jnp.float32)
  outs = pl.pallas_call(
      _heads_tc,
      out_shape=[
          jax.ShapeDtypeStruct((N_FRAMES, 3), jnp.float32),
          jax.ShapeDtypeStruct((N_FRAMES, 3), jnp.float32),
          jax.ShapeDtypeStruct((N_FRAMES, 2), jnp.float32),
          jax.ShapeDtypeStruct((N_SEQS, NUM_SB * 3), jnp.float32),
          jax.ShapeDtypeStruct((N_SEQS, NUM_SB), jnp.float32),
      ],
  )(fp, sp, bbox_center,
    bbox_cos_yaw.reshape(-1, 1), bbox_sin_yaw.reshape(-1, 1), swap,
    Wc, bc.reshape(1, -1), Wv, bv.reshape(1, -1), Wy, by.reshape(1, -1),
    Wsb, bsb.reshape(1, -1), Wsr, bsr.reshape(1, -1))
  centers, velocities, yaw_sincos, size_residual, size_bin = outs
  return (centers, velocities, yaw_sincos, size_residual, size_bin)


# speedup vs baseline: 1.4465x; 1.4465x over previous
import jax
import jax.numpy as jnp
from jax import lax
from jax.experimental import pallas as pl
from jax.experimental.pallas import tpu as pltpu
from jax.experimental.pallas import tpu_sc as plsc

N_POINTS = 160000
N_FRAMES = 10000
N_SEQS = 128
FEAT = 256
NUM_SB = 8
LANE = 16
NFG = FEAT // LANE
NC = 2
NS = 16
NW = NC * NS


def _seg_max_sc(n_items: int, n_out_pad: int, opw: int, ch: int):
  assert n_out_pad == NW * opw

  def body(x_hbm, ids_hbm, bounds_hbm, out_hbm, xbuf, idbuf, bbuf, outbuf):
    w = lax.axis_index("s") * NC + lax.axis_index("c")
    fbase = w * opw
    pltpu.sync_copy(bounds_hbm, bbuf)
    lane = lax.iota(jnp.int32, 16)
    minus1 = jnp.full((16,), -1, jnp.int32)
    ninf = jnp.full((16,), -jnp.inf, jnp.float32)

    def extract(idx):
      r = jnp.int32(-1)
      for k in range(3):
        v = bbuf[pl.ds(k * 16, 16)]
        r = jnp.maximum(r, jnp.max(jnp.where(lane + (k * 16) == idx, v, minus1)))
      return r

    start = extract(w)
    end = extract(w + 1)
    astart = (start // 16) * 16
    nch = jnp.maximum((end - astart + ch - 1) // ch, 0)

    def init_body(i, _):
      outbuf[pl.ds(pl.multiple_of(i * 16, 16), 16)] = ninf
      return 0
    lax.fori_loop(0, opw * NFG, init_body, 0)

    def flush(cur_id, acc):
      roff = pl.multiple_of((cur_id - fbase) * FEAT, FEAT)
      for f in range(NFG):
        o = pl.multiple_of(roff + f * 16, 16)
        outbuf[pl.ds(o, 16)] = jnp.maximum(outbuf[pl.ds(o, 16)], acc[f])

    def chunk_body(c, carry):
      lo = astart + c * ch
      cbase = pl.multiple_of((jnp.minimum(lo, n_items - ch) // 16) * 16, 16)
      pltpu.sync_copy(x_hbm.at[pl.ds(pl.multiple_of(cbase * FEAT, 16), ch * FEAT)],
                      xbuf)
      pltpu.sync_copy(ids_hbm.at[pl.ds(cbase, ch)], idbuf)

      def point_body(j, pcarry):
        cur_id, acc = pcarry
        p = cbase + j
        valid = jnp.logical_and(p >= start, p < end)
        goff = pl.multiple_of((j // 16) * 16, 16)
        idsv = idbuf[pl.ds(goff, 16)]
        idp = jnp.max(jnp.where(lane == (j % 16), idsv, minus1))
        do_flush = jnp.logical_and(valid,
                                   jnp.logical_and(cur_id >= 0, idp != cur_id))

        @pl.when(do_flush)
        def _():
          flush(cur_id, acc)

        same = jnp.logical_and(valid, idp == cur_id)
        first = jnp.logical_and(valid, idp != cur_id)
        xoff = pl.multiple_of(j * FEAT, FEAT)
        nacc = []
        for f in range(NFG):
          row = xbuf[pl.ds(xoff + f * 16, 16)]
          a = jnp.where(same, jnp.maximum(acc[f], row),
                        jnp.where(first, row, acc[f]))
          nacc.append(a)
        ncur = jnp.where(valid, idp, cur_id)
        return ncur, tuple(nacc)

      return lax.fori_loop(0, ch, point_body, carry)

    cur_id, acc = lax.fori_loop(0, nch, chunk_body,
                                (jnp.int32(-1), (ninf,) * NFG))

    @pl.when(cur_id >= 0)
    def _():
      flush(cur_id, acc)

    pltpu.sync_copy(outbuf, out_hbm.at[pl.ds(fbase * FEAT, opw * FEAT)])

  mesh = plsc.VectorSubcoreMesh(core_axis_name="c", subcore_axis_name="s")
  return pl.kernel(
      body,
      out_type=jax.ShapeDtypeStruct((n_out_pad * FEAT,), jnp.float32),
      mesh=mesh,
      scratch_types=[
          pltpu.VMEM((ch * FEAT,), jnp.float32),
          pltpu.VMEM((ch,), jnp.int32),
          pltpu.VMEM((48,), jnp.int32),
          pltpu.VMEM((opw * FEAT,), jnp.float32),
      ],
      compiler_params=pltpu.CompilerParams(needs_layout_passes=False),
  )


def _heads_tc(fp_ref, sp_ref, bbc_ref, cosr_ref, sinr_ref, swap_ref,
              Wc_ref, bc_ref, Wv_ref, bv_ref, Wy_ref, by_ref,
              Wsb_ref, bsb_ref, Wsr_ref, bsr_ref,
              cen_ref, vel_ref, yaw_ref, sr_ref, sb_ref):
  fp = fp_ref[...]
  hp = jax.lax.Precision.HIGHEST
  cen_ref[...] = (jnp.dot(fp, Wc_ref[...], precision=hp,
                          preferred_element_type=jnp.float32)
                  + bc_ref[...] + bbc_ref[...])
  vel_ref[...] = (jnp.dot(fp, Wv_ref[...], precision=hp,
                          preferred_element_type=jnp.float32) + bv_ref[...])
  y = (jnp.dot(fp, Wy_ref[...], precision=hp,
               preferred_element_type=jnp.float32) + by_ref[...])
  ya = jnp.dot(y, swap_ref[...], precision=hp,
               preferred_element_type=jnp.float32)
  yaw_ref[...] = cosr_ref[...] * y + sinr_ref[...] * ya

  @pl.when(pl.program_id(0) == 0)
  def _():
    sp = sp_ref[...]
    sr_ref[...] = (jnp.dot(sp, Wsr_ref[...], precision=hp,
                           preferred_element_type=jnp.float32) + bsr_ref[...])
    logits = (jnp.dot(sp, Wsb_ref[...], precision=hp,
                      preferred_element_type=jnp.float32) + bsb_ref[...])
    m = jnp.max(logits, axis=1, keepdims=True)
    e = jnp.exp(logits - m)
    sb_ref[...] = e / jnp.sum(e, axis=1, keepdims=True)


def kernel(x, bbox_center, bbox_cos_yaw, bbox_sin_yaw, raw_xyz,
           point2frameidx, frame2batchidx,
           Wc, bc, Wy, by, Wv, bv, Wsb, bsb, Wsr, bsr):
  del raw_xyz
  opw1 = 320
  nfp = NW * opw1
  opw2 = N_SEQS // NW

  qs1 = (jnp.arange(NW + 1, dtype=jnp.int32) * opw1).clip(0, N_FRAMES)
  b1 = jnp.searchsorted(point2frameidx, qs1).astype(jnp.int32)
  b1 = jnp.concatenate([b1, jnp.full((48 - (NW + 1),), N_POINTS, jnp.int32)])

  qs2 = jnp.arange(NW + 1, dtype=jnp.int32) * opw2
  b2 = jnp.searchsorted(frame2batchidx, qs2).astype(jnp.int32)
  b2 = jnp.concatenate([b2, jnp.full((48 - (NW + 1),), N_FRAMES, jnp.int32)])

  seg1 = _seg_max_sc(N_POINTS, nfp, opw1, 64)
  fp_flat = seg1(x.reshape(-1), point2frameidx, b1)

  seg2 = _seg_max_sc(N_FRAMES, N_SEQS, opw2, 64)
  sp_flat = seg2(fp_flat, frame2batchidx, b2)

  fp = fp_flat.reshape(nfp, FEAT)[:N_FRAMES]
  sp = sp_flat.reshape(N_SEQS, FEAT)

  swap = jnp.array([[0.0, -1.0], [1.0, 0.0]], dtype=jnp.float32)
  fb = 1000
  ng = N_FRAMES // fb
  row_blk = lambda r: pl.BlockSpec((fb, r), lambda i: (i, 0))
  rep = lambda a, b: pl.BlockSpec((a, b), lambda i: (0, 0))
  outs = pl.pallas_call(
      _heads_tc,
      grid=(ng,),
      in_specs=[
          row_blk(FEAT), rep(N_SEQS, FEAT), row_blk(3), row_blk(1), row_blk(1),
          rep(2, 2),
          rep(FEAT, 3), rep(1, 3), rep(FEAT, 3), rep(1, 3),
          rep(FEAT, 2), rep(1, 2),
          rep(FEAT, NUM_SB), rep(1, NUM_SB),
          rep(FEAT, NUM_SB * 3), rep(1, NUM_SB * 3),
      ],
      out_specs=[
          row_blk(3), row_blk(3), row_blk(2),
          rep(N_SEQS, NUM_SB * 3), rep(N_SEQS, NUM_SB),
      ],
      out_shape=[
          jax.ShapeDtypeStruct((N_FRAMES, 3), jnp.float32),
          jax.ShapeDtypeStruct((N_FRAMES, 3), jnp.float32),
          jax.ShapeDtypeStruct((N_FRAMES, 2), jnp.float32),
          jax.ShapeDtypeStruct((N_SEQS, NUM_SB * 3), jnp.float32),
          jax.ShapeDtypeStruct((N_SEQS, NUM_SB), jnp.float32),
      ],
  )(fp, sp, bbox_center,
    bbox_cos_yaw.reshape(-1, 1), bbox_sin_yaw.reshape(-1, 1), swap,
    Wc, bc.reshape(1, -1), Wv, bv.reshape(1, -1), Wy, by.reshape(1, -1),
    Wsb, bsb.reshape(1, -1), Wsr, bsr.reshape(1, -1))
  centers, velocities, yaw_sincos, size_residual, size_bin = outs
  return (centers, velocities, yaw_sincos, size_residual, size_bin)


# speedup vs baseline: 1.4621x; 1.0108x over previous
import jax
import jax.numpy as jnp
from jax import lax
from jax.experimental import pallas as pl
from jax.experimental.pallas import tpu as pltpu
from jax.experimental.pallas import tpu_sc as plsc

N_POINTS = 160000
N_FRAMES = 10000
N_SEQS = 128
FEAT = 256
NUM_SB = 8
LANE = 16
NFG = FEAT // LANE
NC = 2
NS = 16
NW = NC * NS


def _seg_max_sc(n_items: int, n_out_pad: int, opw: int, ch: int):
  assert n_out_pad == NW * opw

  def body(x_hbm, ids_hbm, bounds_hbm, out_hbm, xbuf, idbuf, bbuf, outbuf):
    w = lax.axis_index("s") * NC + lax.axis_index("c")
    fbase = w * opw
    pltpu.sync_copy(bounds_hbm, bbuf)
    lane = lax.iota(jnp.int32, 16)
    minus1 = jnp.full((16,), -1, jnp.int32)
    ninf = jnp.full((16,), -jnp.inf, jnp.float32)

    def extract(idx):
      r = jnp.int32(-1)
      for k in range(3):
        v = bbuf[pl.ds(k * 16, 16)]
        r = jnp.maximum(r, jnp.max(jnp.where(lane + (k * 16) == idx, v, minus1)))
      return r

    start = extract(w)
    end = extract(w + 1)
    astart = (start // 16) * 16
    nch = jnp.maximum((end - astart + ch - 1) // ch, 0)

    def init_body(i, _):
      outbuf[pl.ds(pl.multiple_of(i * 16, 16), 16)] = ninf
      return 0
    lax.fori_loop(0, opw * NFG, init_body, 0)

    def flush(cur_id, acc):
      roff = pl.multiple_of((cur_id - fbase) * FEAT, FEAT)
      for f in range(NFG):
        o = pl.multiple_of(roff + f * 16, 16)
        outbuf[pl.ds(o, 16)] = jnp.maximum(outbuf[pl.ds(o, 16)], acc[f])

    def chunk_body(c, carry):
      lo = astart + c * ch
      cbase = pl.multiple_of((jnp.minimum(lo, n_items - ch) // 16) * 16, 16)
      pltpu.sync_copy(x_hbm.at[pl.ds(pl.multiple_of(cbase * FEAT, 16), ch * FEAT)],
                      xbuf)
      pltpu.sync_copy(ids_hbm.at[pl.ds(cbase, ch)], idbuf)

      def point_body(j, pcarry):
        cur_id, acc = pcarry
        p = cbase + j
        valid = jnp.logical_and(p >= start, p < end)
        goff = pl.multiple_of((j // 16) * 16, 16)
        idsv = idbuf[pl.ds(goff, 16)]
        idp = jnp.max(jnp.where(lane == (j % 16), idsv, minus1))
        do_flush = jnp.logical_and(valid,
                                   jnp.logical_and(cur_id >= 0, idp != cur_id))

        @pl.when(do_flush)
        def _():
          flush(cur_id, acc)

        same = jnp.logical_and(valid, idp == cur_id)
        first = jnp.logical_and(valid, idp != cur_id)
        xoff = pl.multiple_of(j * FEAT, FEAT)
        nacc = []
        for f in range(NFG):
          row = xbuf[pl.ds(xoff + f * 16, 16)]
          a = jnp.where(same, jnp.maximum(acc[f], row),
                        jnp.where(first, row, acc[f]))
          nacc.append(a)
        ncur = jnp.where(valid, idp, cur_id)
        return ncur, tuple(nacc)

      return lax.fori_loop(0, ch, point_body, carry)

    cur_id, acc = lax.fori_loop(0, nch, chunk_body,
                                (jnp.int32(-1), (ninf,) * NFG))

    @pl.when(cur_id >= 0)
    def _():
      flush(cur_id, acc)

    pltpu.sync_copy(outbuf, out_hbm.at[pl.ds(fbase * FEAT, opw * FEAT)])

  mesh = plsc.VectorSubcoreMesh(core_axis_name="c", subcore_axis_name="s")
  return pl.kernel(
      body,
      out_type=jax.ShapeDtypeStruct((n_out_pad * FEAT,), jnp.float32),
      mesh=mesh,
      scratch_types=[
          pltpu.VMEM((ch * FEAT,), jnp.float32),
          pltpu.VMEM((ch,), jnp.int32),
          pltpu.VMEM((48,), jnp.int32),
          pltpu.VMEM((opw * FEAT,), jnp.float32),
      ],
      compiler_params=pltpu.CompilerParams(needs_layout_passes=False),
  )


def _heads_tc(fp_ref, sp_ref, bbc_ref, cosr_ref, sinr_ref, swap_ref,
              Wc_ref, bc_ref, Wv_ref, bv_ref, Wy_ref, by_ref,
              Wsb_ref, bsb_ref, Wsr_ref, bsr_ref,
              cen_ref, vel_ref, yaw_ref, sr_ref, sb_ref):
  fp = fp_ref[...]
  hp = jax.lax.Precision.HIGHEST
  cen_ref[...] = (jnp.dot(fp, Wc_ref[...], precision=hp,
                          preferred_element_type=jnp.float32)
                  + bc_ref[...] + bbc_ref[...])
  vel_ref[...] = (jnp.dot(fp, Wv_ref[...], precision=hp,
                          preferred_element_type=jnp.float32) + bv_ref[...])
  y = (jnp.dot(fp, Wy_ref[...], precision=hp,
               preferred_element_type=jnp.float32) + by_ref[...])
  ya = jnp.dot(y, swap_ref[...], precision=hp,
               preferred_element_type=jnp.float32)
  yaw_ref[...] = cosr_ref[...] * y + sinr_ref[...] * ya

  @pl.when(pl.program_id(0) == 0)
  def _():
    sp = sp_ref[...]
    sr_ref[...] = (jnp.dot(sp, Wsr_ref[...], precision=hp,
                           preferred_element_type=jnp.float32) + bsr_ref[...])
    logits = (jnp.dot(sp, Wsb_ref[...], precision=hp,
                      preferred_element_type=jnp.float32) + bsb_ref[...])
    m = jnp.max(logits, axis=1, keepdims=True)
    e = jnp.exp(logits - m)
    sb_ref[...] = e / jnp.sum(e, axis=1, keepdims=True)


def kernel(x, bbox_center, bbox_cos_yaw, bbox_sin_yaw, raw_xyz,
           point2frameidx, frame2batchidx,
           Wc, bc, Wy, by, Wv, bv, Wsb, bsb, Wsr, bsr):
  del raw_xyz
  opw1 = 320
  nfp = NW * opw1
  opw2 = N_SEQS // NW

  qs1 = (jnp.arange(NW + 1, dtype=jnp.int32) * opw1).clip(0, N_FRAMES)
  b1 = jnp.searchsorted(point2frameidx, qs1).astype(jnp.int32)
  b1 = jnp.concatenate([b1, jnp.full((48 - (NW + 1),), N_POINTS, jnp.int32)])

  qs2 = jnp.arange(NW + 1, dtype=jnp.int32) * opw2
  b2 = jnp.searchsorted(frame2batchidx, qs2).astype(jnp.int32)
  b2 = jnp.concatenate([b2, jnp.full((48 - (NW + 1),), N_FRAMES, jnp.int32)])

  seg1 = _seg_max_sc(N_POINTS, nfp, opw1, 64)
  fp_flat = seg1(x.reshape(-1), point2frameidx, b1)

  seg2 = _seg_max_sc(N_FRAMES, N_SEQS, opw2, 64)
  sp_flat = seg2(fp_flat, frame2batchidx, b2)

  fp = fp_flat.reshape(nfp, FEAT)
  sp = sp_flat.reshape(N_SEQS, FEAT)
  npad = nfp - N_FRAMES
  bbc_p = jnp.pad(bbox_center, ((0, npad), (0, 0)))
  cos_p = jnp.pad(bbox_cos_yaw.reshape(-1, 1), ((0, npad), (0, 0)))
  sin_p = jnp.pad(bbox_sin_yaw.reshape(-1, 1), ((0, npad), (0, 0)))

  swap = jnp.array([[0.0, -1.0], [1.0, 0.0]], dtype=jnp.float32)
  fb = 1024
  ng = nfp // fb
  row_blk = lambda r: pl.BlockSpec((fb, r), lambda i: (i, 0))
  rep = lambda a, b: pl.BlockSpec((a, b), lambda i: (0, 0))
  outs = pl.pallas_call(
      _heads_tc,
      grid=(ng,),
      in_specs=[
          row_blk(FEAT), rep(N_SEQS, FEAT), row_blk(3), row_blk(1), row_blk(1),
          rep(2, 2),
          rep(FEAT, 3), rep(1, 3), rep(FEAT, 3), rep(1, 3),
          rep(FEAT, 2), rep(1, 2),
          rep(FEAT, NUM_SB), rep(1, NUM_SB),
          rep(FEAT, NUM_SB * 3), rep(1, NUM_SB * 3),
      ],
      out_specs=[
          row_blk(3), row_blk(3), row_blk(2),
          rep(N_SEQS, NUM_SB * 3), rep(N_SEQS, NUM_SB),
      ],
      out_shape=[
          jax.ShapeDtypeStruct((nfp, 3), jnp.float32),
          jax.ShapeDtypeStruct((nfp, 3), jnp.float32),
          jax.ShapeDtypeStruct((nfp, 2), jnp.float32),
          jax.ShapeDtypeStruct((N_SEQS, NUM_SB * 3), jnp.float32),
          jax.ShapeDtypeStruct((N_SEQS, NUM_SB), jnp.float32),
      ],
  )(fp, sp, bbc_p, cos_p, sin_p, swap,
    Wc, bc.reshape(1, -1), Wv, bv.reshape(1, -1), Wy, by.reshape(1, -1),
    Wsb, bsb.reshape(1, -1), Wsr, bsr.reshape(1, -1))
  centers, velocities, yaw_sincos, size_residual, size_bin = outs
  return (centers[:N_FRAMES], velocities[:N_FRAMES], yaw_sincos[:N_FRAMES],
          size_residual, size_bin)


# speedup vs baseline: 1.6796x; 1.1487x over previous
import jax
import jax.numpy as jnp
from jax import lax
from jax.experimental import pallas as pl
from jax.experimental.pallas import tpu as pltpu
from jax.experimental.pallas import tpu_sc as plsc

N_POINTS = 160000
N_FRAMES = 10000
N_SEQS = 128
FEAT = 256
NUM_SB = 8
LANE = 16
NFG = FEAT // LANE
NC = 2
NS = 16
NW = NC * NS


def _seg_max_sc(n_items: int, n_out_pad: int, opw: int, ch: int):
  assert n_out_pad == NW * opw

  def body(x_hbm, ids_hbm, bounds_hbm, out_hbm, xbuf, idbuf, bbuf, outbuf):
    w = lax.axis_index("s") * NC + lax.axis_index("c")
    fbase = w * opw
    pltpu.sync_copy(bounds_hbm, bbuf)
    lane = lax.iota(jnp.int32, 16)
    minus1 = jnp.full((16,), -1, jnp.int32)
    ninf = jnp.full((16,), -jnp.inf, jnp.float32)

    def extract(idx):
      r = jnp.int32(-1)
      for k in range(3):
        v = bbuf[pl.ds(k * 16, 16)]
        r = jnp.maximum(r, jnp.max(jnp.where(lane + (k * 16) == idx, v, minus1)))
      return r

    start = extract(w)
    end = extract(w + 1)
    astart = (start // 16) * 16
    nch = jnp.maximum((end - astart + ch - 1) // ch, 0)

    def init_body(i, _):
      for f in range(NFG):
        outbuf[i, pl.ds(f * 16, 16)] = ninf
      return 0
    lax.fori_loop(0, opw, init_body, 0)

    def flush(cur_id, acc):
      row = cur_id - fbase
      for f in range(NFG):
        outbuf[row, pl.ds(f * 16, 16)] = jnp.maximum(
            outbuf[row, pl.ds(f * 16, 16)], acc[f])

    def chunk_body(c, carry):
      lo = astart + c * ch
      cbase = pl.multiple_of((jnp.minimum(lo, n_items - ch) // 16) * 16, 16)
      pltpu.sync_copy(x_hbm.at[pl.ds(cbase, ch)], xbuf)
      pltpu.sync_copy(ids_hbm.at[pl.ds(cbase, ch)], idbuf)

      def point_body(j, pcarry):
        cur_id, acc = pcarry
        p = cbase + j
        valid = jnp.logical_and(p >= start, p < end)
        goff = pl.multiple_of((j // 16) * 16, 16)
        idsv = idbuf[pl.ds(goff, 16)]
        idp = jnp.max(jnp.where(lane == (j % 16), idsv, minus1))
        do_flush = jnp.logical_and(valid,
                                   jnp.logical_and(cur_id >= 0, idp != cur_id))

        @pl.when(do_flush)
        def _():
          flush(cur_id, acc)

        same = jnp.logical_and(valid, idp == cur_id)
        first = jnp.logical_and(valid, idp != cur_id)
        nacc = []
        for f in range(NFG):
          row = xbuf[j, pl.ds(f * 16, 16)]
          a = jnp.where(same, jnp.maximum(acc[f], row),
                        jnp.where(first, row, acc[f]))
          nacc.append(a)
        ncur = jnp.where(valid, idp, cur_id)
        return ncur, tuple(nacc)

      return lax.fori_loop(0, ch, point_body, carry)

    cur_id, acc = lax.fori_loop(0, nch, chunk_body,
                                (jnp.int32(-1), (ninf,) * NFG))

    @pl.when(cur_id >= 0)
    def _():
      flush(cur_id, acc)

    pltpu.sync_copy(outbuf, out_hbm.at[pl.ds(fbase, opw)])

  mesh = plsc.VectorSubcoreMesh(core_axis_name="c", subcore_axis_name="s")
  return pl.kernel(
      body,
      out_type=jax.ShapeDtypeStruct((n_out_pad, FEAT), jnp.float32),
      mesh=mesh,
      scratch_types=[
          pltpu.VMEM((ch, FEAT), jnp.float32),
          pltpu.VMEM((ch,), jnp.int32),
          pltpu.VMEM((48,), jnp.int32),
          pltpu.VMEM((opw, FEAT), jnp.float32),
      ],
      compiler_params=pltpu.CompilerParams(needs_layout_passes=False),
  )


def _heads_tc(fp_ref, sp_ref, bbc_ref, cosr_ref, sinr_ref, swap_ref,
              Wc_ref, bc_ref, Wv_ref, bv_ref, Wy_ref, by_ref,
              Wsb_ref, bsb_ref, Wsr_ref, bsr_ref,
              cen_ref, vel_ref, yaw_ref, sr_ref, sb_ref):
  fp = fp_ref[...]
  hp = jax.lax.Precision.HIGHEST
  cen_ref[...] = (jnp.dot(fp, Wc_ref[...], precision=hp,
                          preferred_element_type=jnp.float32)
                  + bc_ref[...] + bbc_ref[...])
  vel_ref[...] = (jnp.dot(fp, Wv_ref[...], precision=hp,
                          preferred_element_type=jnp.float32) + bv_ref[...])
  y = (jnp.dot(fp, Wy_ref[...], precision=hp,
               preferred_element_type=jnp.float32) + by_ref[...])
  ya = jnp.dot(y, swap_ref[...], precision=hp,
               preferred_element_type=jnp.float32)
  yaw_ref[...] = cosr_ref[...] * y + sinr_ref[...] * ya

  @pl.when(pl.program_id(0) == 0)
  def _():
    sp = sp_ref[...]
    sr_ref[...] = (jnp.dot(sp, Wsr_ref[...], precision=hp,
                           preferred_element_type=jnp.float32) + bsr_ref[...])
    logits = (jnp.dot(sp, Wsb_ref[...], precision=hp,
                      preferred_element_type=jnp.float32) + bsb_ref[...])
    m = jnp.max(logits, axis=1, keepdims=True)
    e = jnp.exp(logits - m)
    sb_ref[...] = e / jnp.sum(e, axis=1, keepdims=True)


def kernel(x, bbox_center, bbox_cos_yaw, bbox_sin_yaw, raw_xyz,
           point2frameidx, frame2batchidx,
           Wc, bc, Wy, by, Wv, bv, Wsb, bsb, Wsr, bsr):
  del raw_xyz
  opw1 = 320
  nfp = NW * opw1
  opw2 = N_SEQS // NW

  qs1 = (jnp.arange(NW + 1, dtype=jnp.int32) * opw1).clip(0, N_FRAMES)
  b1 = jnp.searchsorted(point2frameidx, qs1).astype(jnp.int32)
  b1 = jnp.concatenate([b1, jnp.full((48 - (NW + 1),), N_POINTS, jnp.int32)])

  qs2 = jnp.arange(NW + 1, dtype=jnp.int32) * opw2
  b2 = jnp.searchsorted(frame2batchidx, qs2).astype(jnp.int32)
  b2 = jnp.concatenate([b2, jnp.full((48 - (NW + 1),), N_FRAMES, jnp.int32)])

  seg1 = _seg_max_sc(N_POINTS, nfp, opw1, 64)
  fp = seg1(x, point2frameidx, b1)

  seg2 = _seg_max_sc(N_FRAMES, N_SEQS, opw2, 64)
  sp = seg2(fp, frame2batchidx, b2)

  npad = nfp - N_FRAMES
  bbc_p = jnp.pad(bbox_center, ((0, npad), (0, 0)))
  cos_p = jnp.pad(bbox_cos_yaw.reshape(-1, 1), ((0, npad), (0, 0)))
  sin_p = jnp.pad(bbox_sin_yaw.reshape(-1, 1), ((0, npad), (0, 0)))

  swap = jnp.array([[0.0, -1.0], [1.0, 0.0]], dtype=jnp.float32)
  fb = 1024
  ng = nfp // fb
  row_blk = lambda r: pl.BlockSpec((fb, r), lambda i: (i, 0))
  rep = lambda a, b: pl.BlockSpec((a, b), lambda i: (0, 0))
  outs = pl.pallas_call(
      _heads_tc,
      grid=(ng,),
      in_specs=[
          row_blk(FEAT), rep(N_SEQS, FEAT), row_blk(3), row_blk(1), row_blk(1),
          rep(2, 2),
          rep(FEAT, 3), rep(1, 3), rep(FEAT, 3), rep(1, 3),
          rep(FEAT, 2), rep(1, 2),
          rep(FEAT, NUM_SB), rep(1, NUM_SB),
          rep(FEAT, NUM_SB * 3), rep(1, NUM_SB * 3),
      ],
      out_specs=[
          row_blk(3), row_blk(3), row_blk(2),
          rep(N_SEQS, NUM_SB * 3), rep(N_SEQS, NUM_SB),
      ],
      out_shape=[
          jax.ShapeDtypeStruct((nfp, 3), jnp.float32),
          jax.ShapeDtypeStruct((nfp, 3), jnp.float32),
          jax.ShapeDtypeStruct((nfp, 2), jnp.float32),
          jax.ShapeDtypeStruct((N_SEQS, NUM_SB * 3), jnp.float32),
          jax.ShapeDtypeStruct((N_SEQS, NUM_SB), jnp.float32),
      ],
  )(fp, sp, bbc_p, cos_p, sin_p, swap,
    Wc, bc.reshape(1, -1), Wv, bv.reshape(1, -1), Wy, by.reshape(1, -1),
    Wsb, bsb.reshape(1, -1), Wsr, bsr.reshape(1, -1))
  centers, velocities, yaw_sincos, size_residual, size_bin = outs
  return (centers[:N_FRAMES], velocities[:N_FRAMES], yaw_sincos[:N_FRAMES],
          size_residual, size_bin)
